# Initial kernel scaffold; baseline (speedup 1.0000x reference)
import jax
import jax.numpy as jnp
from jax.experimental import pallas as pl


def kernel(features, adj, target_len, neighbor_len, target_index_out, W, a):
    raise NotImplementedError("write your pallas kernel here")


# speedup vs baseline: 1.0595x; 1.0595x over previous
import functools

import jax
import jax.numpy as jnp
from jax import lax
from jax.experimental import pallas as pl
from jax.experimental.pallas import tpu as pltpu
from jax.experimental.pallas import tpu_sc as plsc

N = 10000
E = 160000
D = 128
T = 1024
ALPHA = 0.2
FILL = -9000000000000000.0

NUM_TILES = 32
ROWS_PER_TILE = T // NUM_TILES
SUB_ROWS = 2
NUM_SUB = ROWS_PER_TILE // SUB_ROWS
CS = 4000
NCHUNK = E // CS
CAP = 8192
L = 16


def _hs_body(f_ref, w_ref, a1_ref, a2_ref, h_ref, s_ref):
    h = jnp.dot(f_ref[...], w_ref[...], preferred_element_type=jnp.float32)
    h_ref[...] = h
    s1 = jnp.sum(h * a1_ref[...], axis=1, keepdims=True)
    s2 = jnp.sum(h * a2_ref[...], axis=1, keepdims=True)
    s_ref[...] = jnp.concatenate([s1, s2], axis=1)


def _compute_h_s(features, W, a):
    a1 = a[:D, :].reshape(1, D)
    a2 = a[D:, :].reshape(1, D)
    blk = 1000
    return pl.pallas_call(
        _hs_body,
        grid=(N // blk,),
        in_specs=[
            pl.BlockSpec((blk, D), lambda i: (i, 0)),
            pl.BlockSpec((D, D), lambda i: (0, 0)),
            pl.BlockSpec((1, D), lambda i: (0, 0)),
            pl.BlockSpec((1, D), lambda i: (0, 0)),
        ],
        out_specs=[
            pl.BlockSpec((blk, D), lambda i: (i, 0)),
            pl.BlockSpec((blk, 2), lambda i: (i, 0)),
        ],
        out_shape=[
            jax.ShapeDtypeStruct((N, D), jnp.float32),
            jax.ShapeDtypeStruct((N, 2), jnp.float32),
        ],
    )(features, W, a1, a2)


def _sm_body(att_ref, h_ref, o_ref):
    att = att_ref[...]
    m = jnp.max(att, axis=1, keepdims=True)
    p = jnp.exp(att - m)
    z = jnp.sum(p, axis=1, keepdims=True)
    o = jnp.dot(p / z, h_ref[...], preferred_element_type=jnp.float32)
    o_ref[...] = jnp.where(o > 0, o, jnp.exp(o) - 1.0)


def _softmax_matmul(att, h):
    rb = 64
    return pl.pallas_call(
        _sm_body,
        grid=(T // rb,),
        in_specs=[
            pl.BlockSpec((rb, N), lambda i: (i, 0)),
            pl.BlockSpec((N, D), lambda i: (0, 0)),
        ],
        out_specs=pl.BlockSpec((rb, D), lambda i: (i, 0)),
        out_shape=jax.ShapeDtypeStruct((T, D), jnp.float32),
    )(att, h)


def _scatter_tile(tio_hbm, a0_hbm, a1_hbm, win_hbm, s1_hbm, s2_hbm,
                  att_hbm, fill_hbm,
                  s1_v, s2_v, tio_c, a0_c, a1_c,
                  my_t, my_a0, my_a1, my_idf, my_e,
                  fill_row, winband, rowbuf):
    wid = lax.axis_index("s") * 2 + lax.axis_index("c")
    lo = wid * ROWS_PER_TILE
    hi = lo + ROWS_PER_TILE

    pltpu.sync_copy(s1_hbm, s1_v)
    pltpu.sync_copy(s2_hbm, s2_v)

    def init_fill(i, _):
        fill_row[pl.ds(i * L, L)] = jnp.full((L,), FILL, jnp.float32)
        return 0
    lax.fori_loop(0, N // L, init_fill, 0)
    pltpu.sync_copy(fill_row, fill_hbm)

    lane_iota = lax.iota(jnp.int32, L)

    def scan_chunk(c, off):
        base = c * CS
        pltpu.sync_copy(tio_hbm.at[pl.ds(base, CS)], tio_c)
        pltpu.sync_copy(a0_hbm.at[pl.ds(base, CS)], a0_c)
        pltpu.sync_copy(a1_hbm.at[pl.ds(base, CS)], a1_c)

        def vec_iter(i, off):
            t = tio_c[pl.ds(i * L, L)]
            m = (t >= lo) & (t < hi)
            cnt = jnp.sum(jnp.where(m, 1, 0).astype(jnp.int32), axis=0)

            @pl.when(cnt > 0)
            def _():
                plsc.store_compressed(my_t.at[pl.ds(off, L)], t, mask=m)
                plsc.store_compressed(my_a0.at[pl.ds(off, L)],
                                      a0_c[pl.ds(i * L, L)], mask=m)
                plsc.store_compressed(my_a1.at[pl.ds(off, L)],
                                      a1_c[pl.ds(i * L, L)], mask=m)
                idv = (base + i * L + lane_iota).astype(jnp.float32)
                plsc.store_compressed(my_idf.at[pl.ds(off, L)], idv, mask=m)
            return off + cnt
        return lax.fori_loop(0, CS // L, vec_iter, off)

    nm = lax.fori_loop(0, NCHUNK, scan_chunk, jnp.int32(0))

    my_t[pl.ds(nm, L)] = jnp.full((L,), 2 * T, jnp.int32)
    my_a0[pl.ds(nm, L)] = jnp.zeros((L,), jnp.int32)
    my_a1[pl.ds(nm, L)] = jnp.zeros((L,), jnp.int32)
    my_idf[pl.ds(nm, L)] = jnp.full((L,), -1.0, jnp.float32)
    n_vec = (nm + L - 1) // L

    def e_iter(i, _):
        g1 = plsc.load_gather(s1_v, [my_a0[pl.ds(i * L, L)]])
        g2 = plsc.load_gather(s2_v, [my_a1[pl.ds(i * L, L)]])
        e = g1 + g2
        my_e[pl.ds(i * L, L)] = jnp.where(e > 0, e, ALPHA * e)
        return 0
    lax.fori_loop(0, n_vec, e_iter, 0)

    def sub_iter(s, _):
        row0 = lo + s * SUB_ROWS
        pltpu.sync_copy(win_hbm.at[pl.ds(row0, SUB_ROWS)], winband)

        def memset_row(r, _):
            pltpu.sync_copy(fill_hbm, rowbuf.at[r])
            return 0
        lax.fori_loop(0, SUB_ROWS, memset_row, 0)

        def sc_iter(i, _):
            sl = pl.ds(i * L, L)
            t = my_t[sl]
            m = (t >= row0) & (t < row0 + SUB_ROWS)
            r = jnp.where(m, t - row0, 0)
            col = my_a1[sl]
            wv = plsc.load_gather(winband, [r, col], mask=m)
            keep = m & (wv == my_idf[sl])
            plsc.store_scatter(rowbuf, [r, col], my_e[sl], mask=keep)
            return 0
        lax.fori_loop(0, n_vec, sc_iter, 0)

        pltpu.sync_copy(rowbuf, att_hbm.at[pl.ds(row0, SUB_ROWS)])
        return 0
    lax.fori_loop(0, NUM_SUB, sub_iter, 0)


def _build_attention(tio, adj0, adj1, win, s1, s2):
    mesh = plsc.VectorSubcoreMesh(core_axis_name="c", subcore_axis_name="s")
    f = functools.partial(
        pl.kernel,
        out_type=(
            jax.ShapeDtypeStruct((T, N), jnp.float32),
            jax.ShapeDtypeStruct((N,), jnp.float32),
        ),
        mesh=mesh,
        compiler_params=pltpu.CompilerParams(needs_layout_passes=False),
        scratch_types=[
            pltpu.VMEM((N,), jnp.float32),
            pltpu.VMEM((N,), jnp.float32),
            pltpu.VMEM((CS,), jnp.int32),
            pltpu.VMEM((CS,), jnp.int32),
            pltpu.VMEM((CS,), jnp.int32),
            pltpu.VMEM((CAP + L,), jnp.int32),
            pltpu.VMEM((CAP + L,), jnp.int32),
            pltpu.VMEM((CAP + L,), jnp.int32),
            pltpu.VMEM((CAP + L,), jnp.float32),
            pltpu.VMEM((CAP + L,), jnp.float32),
            pltpu.VMEM((N,), jnp.float32),
            pltpu.VMEM((SUB_ROWS, N), jnp.float32),
            pltpu.VMEM((SUB_ROWS, N), jnp.float32),
        ],
    )(_scatter_tile)
    att, _ = f(tio, adj0, adj1, win, s1, s2)
    return att


def kernel(features, adj, target_len, neighbor_len, target_index_out, W, a):
    h, s = _compute_h_s(features, W, a)
    ids = jnp.arange(E, dtype=jnp.float32)
    win = jnp.full((T, N), FILL, dtype=jnp.float32)
    win = win.at[target_index_out, adj[1]].set(ids)
    att = _build_attention(target_index_out, adj[0], adj[1], win,
                           s[:, 0], s[:, 1])
    return _softmax_matmul(att, h)


# speedup vs baseline: 1.4715x; 1.3889x over previous
import functools

import jax
import jax.numpy as jnp
from jax import lax
from jax.experimental import pallas as pl
from jax.experimental.pallas import tpu as pltpu
from jax.experimental.pallas import tpu_sc as plsc

N = 10000
E = 160000
D = 128
T = 1024
ALPHA = 0.2
FILL = -9000000000000000.0

NT = 16
RPT = T // NT
SL = E // NT
CS = 2000
AC = 768
BC = 256
L = 16


def _hs_body(f_ref, w_ref, a1_ref, a2_ref, h_ref, s_ref):
    h = jnp.dot(f_ref[...], w_ref[...], preferred_element_type=jnp.float32)
    h_ref[...] = h
    s1 = jnp.sum(h * a1_ref[...], axis=1, keepdims=True)
    s2 = jnp.sum(h * a2_ref[...], axis=1, keepdims=True)
    s_ref[...] = jnp.concatenate([s1, s2], axis=1)


def _compute_h_s(features, W, a):
    a1 = a[:D, :].reshape(1, D)
    a2 = a[D:, :].reshape(1, D)
    blk = 1000
    return pl.pallas_call(
        _hs_body,
        grid=(N // blk,),
        in_specs=[
            pl.BlockSpec((blk, D), lambda i: (i, 0)),
            pl.BlockSpec((D, D), lambda i: (0, 0)),
            pl.BlockSpec((1, D), lambda i: (0, 0)),
            pl.BlockSpec((1, D), lambda i: (0, 0)),
        ],
        out_specs=[
            pl.BlockSpec((blk, D), lambda i: (i, 0)),
            pl.BlockSpec((blk, 2), lambda i: (i, 0)),
        ],
        out_shape=[
            jax.ShapeDtypeStruct((N, D), jnp.float32),
            jax.ShapeDtypeStruct((N, 2), jnp.float32),
        ],
    )(features, W, a1, a2)


def _sm_body(att_ref, h_ref, o_ref):
    att = att_ref[...]
    m = jnp.max(att, axis=1, keepdims=True)
    p = jnp.exp(att - m)
    z = jnp.sum(p, axis=1, keepdims=True)
    o = jnp.dot(p / z, h_ref[...], preferred_element_type=jnp.float32)
    o_ref[...] = jnp.where(o > 0, o, jnp.exp(o) - 1.0)


def _softmax_matmul(att, h):
    rb = 64
    return pl.pallas_call(
        _sm_body,
        grid=(T // rb,),
        in_specs=[
            pl.BlockSpec((rb, N), lambda i: (i, 0)),
            pl.BlockSpec((N, D), lambda i: (0, 0)),
        ],
        out_specs=pl.BlockSpec((rb, D), lambda i: (i, 0)),
        out_shape=jax.ShapeDtypeStruct((T, D), jnp.float32),
    )(att, h)


def _scatter_tile(tio_hbm, a0_hbm, a1_hbm, win_hbm, s1_hbm, s2_hbm,
                  att_hbm, stg_ta_hbm, stg_id_hbm, stg_e_hbm, counts_hbm,
                  s1_v, s2_v, tio_c, a0_c, a1_c,
                  a_ta, a_id, a_e, off_a,
                  sb_ta, sb_id, sb_e, counts_v,
                  b_col, b_id, b_e, off_b, winband):
    wid = lax.axis_index("s") + 0 * lax.axis_index("c")
    lo = wid * RPT
    lane_iota = lax.iota(jnp.int32, L)

    pltpu.sync_copy(s1_hbm, s1_v)
    pltpu.sync_copy(s2_hbm, s2_v)
    off_a[...] = jnp.zeros((L,), jnp.int32)

    def scan_chunk(c, _):
        base = wid * SL + c * CS
        pltpu.sync_copy(tio_hbm.at[pl.ds(base, CS)], tio_c)
        pltpu.sync_copy(a0_hbm.at[pl.ds(base, CS)], a0_c)
        pltpu.sync_copy(a1_hbm.at[pl.ds(base, CS)], a1_c)

        def vec_iter(i, _):
            sl = pl.ds(i * L, L)
            t = tio_c[sl]
            a1v = a1_c[sl]
            g1 = plsc.load_gather(s1_v, [a0_c[sl]])
            g2 = plsc.load_gather(s2_v, [a1v])
            ev = g1 + g2
            ev = jnp.where(ev > 0, ev, ALPHA * ev)
            idv = (base + i * L + lane_iota).astype(jnp.float32)
            b = t >> 6
            rank, lastm = plsc.scan_count(b)
            basev = plsc.load_gather(off_a, [b])
            pos = basev + rank - 1
            addr = b * AC + pos
            plsc.store_scatter(a_ta, [addr], t * 16384 + a1v)
            plsc.store_scatter(a_id, [addr], idv)
            plsc.store_scatter(a_e, [addr], ev)
            plsc.store_scatter(off_a, [b], pos + 1, mask=lastm)
            return 0
        lax.fori_loop(0, CS // L, vec_iter, 0)
        return 0
    lax.fori_loop(0, SL // CS, scan_chunk, 0)

    pltpu.sync_copy(a_ta, stg_ta_hbm.at[wid])
    pltpu.sync_copy(a_id, stg_id_hbm.at[wid])
    pltpu.sync_copy(a_e, stg_e_hbm.at[wid])
    pltpu.sync_copy(off_a, counts_hbm.at[pl.ds(wid * L, L)])
    plsc.subcore_barrier()

    pltpu.sync_copy(counts_hbm, counts_v.at[pl.ds(0, NT * L)])
    for j in range(0, RPT + L, L):
        off_b[pl.ds(j, L)] = jnp.zeros((L,), jnp.int32)

    def src_iter(j, _):
        pltpu.sync_copy(stg_ta_hbm.at[j, pl.ds(wid * AC, AC)], sb_ta)
        pltpu.sync_copy(stg_id_hbm.at[j, pl.ds(wid * AC, AC)], sb_id)
        pltpu.sync_copy(stg_e_hbm.at[j, pl.ds(wid * AC, AC)], sb_e)
        cntv = plsc.load_gather(counts_v,
                                [jnp.full((L,), j * L + wid, jnp.int32)])

        def vec_iter(i, _):
            sl = pl.ds(i * L, L)
            mv = (i * L + lane_iota) < cntv
            ta = sb_ta[sl]
            t = ta >> 14
            col = ta & 16383
            bk = jnp.where(mv, t - lo, RPT)
            rank, lastm = plsc.scan_count(bk)
            basev = plsc.load_gather(off_b, [bk])
            pos = basev + rank - 1
            addr = bk * BC + pos
            plsc.store_scatter(b_col, [addr], col)
            plsc.store_scatter(b_id, [addr], sb_id[sl])
            plsc.store_scatter(b_e, [addr], sb_e[sl])
            plsc.store_scatter(off_b, [bk], pos + 1, mask=lastm & mv)
            return 0
        lax.fori_loop(0, AC // L, vec_iter, 0)
        return 0
    lax.fori_loop(0, NT, src_iter, 0)

    def row_iter(bk, _):
        row = lo + bk
        pltpu.sync_copy(win_hbm.at[row], winband)
        cntv = plsc.load_gather(off_b, [jnp.full((L,), bk, jnp.int32)])
        base = bk * BC

        def pass1(i, _):
            sl = pl.ds(base + i * L, L)
            mv = (i * L + lane_iota) < cntv
            col = jnp.where(mv, b_col[sl], 0)
            wv = plsc.load_gather(winband, [col], mask=mv)
            keep = mv & (wv == b_id[sl])
            b_id[sl] = jnp.where(keep, b_id[sl], -3.0)
            return 0
        lax.fori_loop(0, BC // L, pass1, 0)


        def pass2(i, _):
            sl = pl.ds(base + i * L, L)
            mv = (i * L + lane_iota) < cntv
            keep = mv & (b_id[sl] >= 0)
            col = jnp.where(mv, b_col[sl], 0)
            plsc.store_scatter(winband, [col], b_e[sl], mask=keep)
            return 0
        lax.fori_loop(0, BC // L, pass2, 0)

        pltpu.sync_copy(winband, att_hbm.at[row])
        return 0
    lax.fori_loop(0, RPT, row_iter, 0)


def _build_attention(tio, adj0, adj1, win, s1, s2):
    mesh = plsc.VectorSubcoreMesh(core_axis_name="c", subcore_axis_name="s",
                                  num_cores=1)
    f = functools.partial(
        pl.kernel,
        out_type=(
            jax.ShapeDtypeStruct((T, N), jnp.float32),
            jax.ShapeDtypeStruct((NT, NT * AC), jnp.int32),
            jax.ShapeDtypeStruct((NT, NT * AC), jnp.float32),
            jax.ShapeDtypeStruct((NT, NT * AC), jnp.float32),
            jax.ShapeDtypeStruct((NT * L,), jnp.int32),
        ),
        mesh=mesh,
        compiler_params=pltpu.CompilerParams(needs_layout_passes=False),
        scratch_types=[
            pltpu.VMEM((N,), jnp.float32),
            pltpu.VMEM((N,), jnp.float32),
            pltpu.VMEM((CS,), jnp.int32),
            pltpu.VMEM((CS,), jnp.int32),
            pltpu.VMEM((CS,), jnp.int32),
            pltpu.VMEM((NT * AC,), jnp.int32),
            pltpu.VMEM((NT * AC,), jnp.float32),
            pltpu.VMEM((NT * AC,), jnp.float32),
            pltpu.VMEM((L,), jnp.int32),
            pltpu.VMEM((AC,), jnp.int32),
            pltpu.VMEM((AC,), jnp.float32),
            pltpu.VMEM((AC,), jnp.float32),
            pltpu.VMEM((NT * L + L,), jnp.int32),
            pltpu.VMEM(((RPT + 1) * BC,), jnp.int32),
            pltpu.VMEM(((RPT + 1) * BC,), jnp.float32),
            pltpu.VMEM(((RPT + 1) * BC,), jnp.float32),
            pltpu.VMEM((RPT + 1 + L,), jnp.int32),
            pltpu.VMEM((N,), jnp.float32),
        ],
    )(_scatter_tile)
    att = f(tio, adj0, adj1, win, s1, s2)[0]
    return att


def kernel(features, adj, target_len, neighbor_len, target_index_out, W, a):
    h, s = _compute_h_s(features, W, a)
    ids = jnp.arange(E, dtype=jnp.float32)
    win = jnp.full((T, N), FILL, dtype=jnp.float32)
    win = win.at[target_index_out, adj[1]].set(ids)
    att = _build_attention(target_index_out, adj[0], adj[1], win,
                           s[:, 0], s[:, 1])
    return _softmax_matmul(att, h)
